# baseline (speedup 1.0000x reference)
import jax
import jax.numpy as jnp
from jax.experimental import pallas as pl
from jax.experimental.pallas import tpu as pltpu

_f32 = jnp.float32
_bf16 = jnp.bfloat16

_TAPS = [(dy, dx) for dy in (-1, 0, 1) for dx in (-1, 0, 1)]


def _make_upconv_kernel(H, W):
    def body(ll_ref, lh_ref, hl_ref, hh_ref, w_ref, b_ref, y_ref, stats_ref):
        Cin = ll_ref.shape[1]
        Cout = w_ref.shape[2]

        def to_cl(ref, relu):
            x = jnp.transpose(ref[0], (1, 0))
            if relu:
                x = jnp.maximum(x, 0.0)
            return x.reshape(H, W, Cin)

        a = to_cl(ll_ref, True)
        b = to_cl(lh_ref, False)
        c = to_cl(hl_ref, False)
        d = to_cl(hh_ref, False)

        p00 = 0.5 * (a - b - c + d)
        p01 = 0.5 * (a + b - c - d)
        p10 = 0.5 * (a - b + c - d)
        p11 = 0.5 * (a + b + c + d)

        HW = H * W
        flat = lambda x: x.reshape(HW, Cin)
        col = jax.lax.broadcasted_iota(jnp.int32, (HW, Cin), 0) % W
        zrow = jnp.zeros((1, Cin), _f32)
        zpad = jnp.zeros((W, Cin), _bf16)

        def padded(x):
            return jnp.concatenate([zpad, x.astype(_bf16), zpad], axis=0)

        def shift_pos(x):
            s = jnp.concatenate([x[1:], zrow], axis=0)
            return jnp.where(col == W - 1, 0.0, s)

        def shift_neg(x):
            s = jnp.concatenate([zrow, x[:-1]], axis=0)
            return jnp.where(col == 0, 0.0, s)

        p00, p01, p10, p11 = map(flat, (p00, p01, p10, p11))
        base = {
            (0, 0, 0): padded(p00), (0, 0, 1): padded(shift_pos(p00)),
            (1, 0, 0): padded(p10), (1, 0, 1): padded(shift_pos(p10)),
            (0, 1, 0): padded(p01), (0, 1, -1): padded(shift_neg(p01)),
            (1, 1, 0): padded(p11), (1, 1, -1): padded(shift_neg(p11)),
        }

        ones = jnp.ones((1, HW), _f32)
        accs = {}
        s1 = jnp.zeros((1, Cout), _f32)
        s2 = jnp.zeros((1, Cout), _f32)
        for p in (0, 1):
            for q in (0, 1):
                acc = jnp.broadcast_to(b_ref[...], (HW, Cout))
                for k, (dy, dx) in enumerate(_TAPS):
                    s = (p + dy) & 1
                    t = (q + dx) & 1
                    dlt = (p + dy) >> 1
                    gam = (q + dx) >> 1
                    op = base[(s, t, gam)][(dlt + 1) * W:(dlt + 1) * W + HW]
                    acc = acc + jnp.dot(op, w_ref[k],
                                        preferred_element_type=_f32)
                s1 = s1 + jnp.dot(ones, acc, preferred_element_type=_f32)
                s2 = s2 + jnp.dot(ones, acc * acc, preferred_element_type=_f32)
                accs[(p, q)] = acc.reshape(H, W, Cout)

        even = jnp.stack([accs[(0, 0)], accs[(0, 1)]], axis=2)
        odd = jnp.stack([accs[(1, 0)], accs[(1, 1)]], axis=2)
        full = jnp.stack([even, odd], axis=1)
        y_ref[0] = full.reshape(4 * H * W, Cout).astype(_bf16)

        stats_ref[0, 0:1, :] = s1
        stats_ref[0, 1:2, :] = s2

    return body


def _bn_apply_kernel(y_ref, scale_ref, shift_ref, o_ref):
    y = y_ref[0].astype(_f32) * scale_ref[...] + shift_ref[...]
    o_ref[0] = jnp.transpose(y, (1, 0))


def kernel(LL, LH, HL, HH, conv_w, conv_b, bn_gamma, bn_beta, *, eps=1e-5):
    N, Cin, H, W = LL.shape
    Cout = conv_w.shape[0]
    OHW = 4 * H * W

    flat = lambda x: x.reshape(N, Cin, H * W)
    ll, lh, hl, hh = map(flat, (LL, LH, HL, HH))

    w2 = jnp.transpose(conv_w, (2, 3, 1, 0)).reshape(9, Cin, Cout).astype(_bf16)
    b2 = conv_b.reshape(1, Cout).astype(_f32)

    band_spec = pl.BlockSpec((1, Cin, H * W), lambda n: (n, 0, 0))

    y, stats = pl.pallas_call(
        _make_upconv_kernel(H, W),
        out_shape=(jax.ShapeDtypeStruct((N, OHW, Cout), _bf16),
                   jax.ShapeDtypeStruct((N, 2, Cout), _f32)),
        grid_spec=pltpu.PrefetchScalarGridSpec(
            num_scalar_prefetch=0,
            grid=(N,),
            in_specs=[band_spec, band_spec, band_spec, band_spec,
                      pl.BlockSpec((9, Cin, Cout), lambda n: (0, 0, 0)),
                      pl.BlockSpec((1, Cout), lambda n: (0, 0))],
            out_specs=(pl.BlockSpec((1, OHW, Cout), lambda n: (n, 0, 0)),
                       pl.BlockSpec((1, 2, Cout), lambda n: (n, 0, 0)))),
        compiler_params=pltpu.CompilerParams(
            dimension_semantics=("parallel",)),
    )(ll, lh, hl, hh, w2, b2)

    cnt = float(N * OHW)
    s = jnp.sum(stats, axis=0)
    mean = s[0] / cnt
    var = jnp.maximum(s[1] / cnt - mean * mean, 0.0)
    scale = (bn_gamma.astype(_f32) * jax.lax.rsqrt(var + eps)).reshape(1, Cout)
    shift = bn_beta.astype(_f32).reshape(1, Cout) - mean.reshape(1, Cout) * scale

    R2 = 8 if OHW % 8 == 0 else 1
    L2 = OHW // R2
    y_bn = pl.pallas_call(
        _bn_apply_kernel,
        out_shape=jax.ShapeDtypeStruct((N, Cout, OHW), _f32),
        grid_spec=pltpu.PrefetchScalarGridSpec(
            num_scalar_prefetch=0,
            grid=(N, R2),
            in_specs=[pl.BlockSpec((1, L2, Cout), lambda n, r: (n, r, 0)),
                      pl.BlockSpec((1, Cout), lambda n, r: (0, 0)),
                      pl.BlockSpec((1, Cout), lambda n, r: (0, 0))],
            out_specs=pl.BlockSpec((1, Cout, L2), lambda n, r: (n, 0, r))),
        compiler_params=pltpu.CompilerParams(
            dimension_semantics=("parallel", "parallel")),
    )(y, scale, shift)

    return y_bn.reshape(N, Cout, 2 * H, 2 * W)


# speedup vs baseline: 1.7367x; 1.7367x over previous
import jax
import jax.numpy as jnp
from jax.experimental import pallas as pl
from jax.experimental.pallas import tpu as pltpu

_f32 = jnp.float32
_bf16 = jnp.bfloat16


def _make_upconv_kernel(H, W, TH):
    def body(ll_ref, lh_ref, hl_ref, hh_ref, wp_ref, b_ref, y_ref, stats_ref):
        Cin = ll_ref.shape[1]
        Cout = wp_ref.shape[4]
        M = TH * W
        SL = (TH + 2) * W

        r = pl.program_id(1)
        nrb = pl.num_programs(1)
        r0 = pl.multiple_of(r * TH, TH)

        def pair_cl(refa, refb, start, n, relu):
            va = refa[0, :, pl.ds(start, n), :].reshape(Cin, n * W)
            if relu:
                va = jnp.maximum(va, 0.0)
            vb = refb[0, :, pl.ds(start, n), :].reshape(Cin, n * W)
            return jnp.transpose(jnp.concatenate([va, vb], axis=0), (1, 0))

        tmask = (r > 0).astype(_f32)
        bmask = (r < nrb - 1).astype(_f32)
        top = jnp.maximum(r0 - 1, 0)
        bot = jnp.minimum(r0 + TH, H - 1)

        def slab(refa, refb, relu):
            t = pair_cl(refa, refb, top, 1, relu) * tmask
            m = pair_cl(refa, refb, r0, TH, relu)
            b = pair_cl(refa, refb, bot, 1, relu) * bmask
            return jnp.concatenate([t, m, b], axis=0)

        t1 = slab(ll_ref, lh_ref, True)
        t2 = slab(hl_ref, hh_ref, False)

        lane = jax.lax.broadcasted_iota(jnp.int32, (SL, 2 * Cin), 1)
        sgn = jnp.where(lane < Cin, -1.0, 1.0).astype(_f32)

        def rot(x):
            return jnp.concatenate([x[:, Cin:], x[:, :Cin]], axis=1)

        u = t1 - t2
        v = t1 + t2
        p0 = (0.5 * (u + sgn * rot(u))).astype(_bf16)
        p1 = (0.5 * (v + sgn * rot(v))).astype(_bf16)

        col = jax.lax.broadcasted_iota(jnp.int32, (SL, 2 * Cin), 0) % W
        ml = ((lane < Cin) & (col != W - 1)).astype(_bf16)
        mr = ((lane >= Cin) & (col != 0)).astype(_bf16)
        zrow = jnp.zeros((1, 2 * Cin), _bf16)

        def shifted(x):
            sp = jnp.concatenate([x[1:], zrow], axis=0)
            sn = jnp.concatenate([zrow, x[:-1]], axis=0)
            return sp * ml + sn * mr

        ps0 = shifted(p0)
        ps1 = shifted(p1)
        pb = (p0, p1)
        psb = (ps0, ps1)

        accs = []
        for p in (0, 1):
            for q in (0, 1):
                acc = jnp.broadcast_to(b_ref[...], (M, Cout))
                for dy in (-1, 0, 1):
                    s = (p + dy) & 1
                    beg = ((p + dy) >> 1) * W + W
                    acc = acc + jnp.dot(pb[s][beg:beg + M],
                                        wp_ref[q, dy + 1, 0],
                                        preferred_element_type=_f32)
                    acc = acc + jnp.dot(psb[s][beg:beg + M],
                                        wp_ref[q, dy + 1, 1],
                                        preferred_element_type=_f32)
                accs.append(acc.reshape(TH, W, Cout))

        even = jnp.stack([accs[0], accs[1]], axis=2)
        odd = jnp.stack([accs[2], accs[3]], axis=2)
        full = jnp.stack([even, odd], axis=1)
        fullb = full.reshape(4 * M, Cout).astype(_bf16)
        y_ref[0] = fullb

        ones = jnp.ones((1, 4 * M), _bf16)
        stats_ref[0, 0, 0:1, :] = jnp.dot(ones, fullb,
                                          preferred_element_type=_f32)
        stats_ref[0, 0, 1:2, :] = jnp.dot(ones, fullb * fullb,
                                          preferred_element_type=_f32)

    return body


def _make_bn_kernel(RG, W2):
    def body(y_ref, scale_ref, shift_ref, o_ref):
        y = y_ref[0].astype(_f32) * scale_ref[...] + shift_ref[...]
        t = jnp.transpose(y, (1, 0))
        o_ref[0] = t.reshape(t.shape[0], RG, W2)

    return body


def kernel(LL, LH, HL, HH, conv_w, conv_b, bn_gamma, bn_beta, *, eps=1e-5):
    N, Cin, H, W = LL.shape
    Cout = conv_w.shape[0]
    OHW = 4 * H * W
    H2, W2 = 2 * H, 2 * W
    TH = 16 if H % 16 == 0 else H
    R = H // TH

    wt = jnp.transpose(conv_w, (2, 3, 1, 0))
    z = jnp.zeros((3, Cin, Cout), conv_w.dtype)
    wp = jnp.stack([
        jnp.stack([jnp.concatenate([wt[:, 1], wt[:, 2]], axis=1),
                   jnp.concatenate([z, wt[:, 0]], axis=1)], axis=1),
        jnp.stack([jnp.concatenate([wt[:, 0], wt[:, 1]], axis=1),
                   jnp.concatenate([wt[:, 2], z], axis=1)], axis=1),
    ], axis=0).astype(_bf16)
    b2 = conv_b.reshape(1, Cout).astype(_f32)

    band_spec = pl.BlockSpec((1, Cin, H, W), lambda n, r: (n, 0, 0, 0))

    y, stats = pl.pallas_call(
        _make_upconv_kernel(H, W, TH),
        out_shape=(jax.ShapeDtypeStruct((N, OHW, Cout), _bf16),
                   jax.ShapeDtypeStruct((N, R, 2, Cout), _f32)),
        grid_spec=pltpu.PrefetchScalarGridSpec(
            num_scalar_prefetch=0,
            grid=(N, R),
            in_specs=[band_spec, band_spec, band_spec, band_spec,
                      pl.BlockSpec((2, 3, 2, 2 * Cin, Cout),
                                   lambda n, r: (0, 0, 0, 0, 0)),
                      pl.BlockSpec((1, Cout), lambda n, r: (0, 0))],
            out_specs=(pl.BlockSpec((1, 4 * TH * W, Cout),
                                    lambda n, r: (n, r, 0)),
                       pl.BlockSpec((1, 1, 2, Cout),
                                    lambda n, r: (n, r, 0, 0)))),
        compiler_params=pltpu.CompilerParams(
            dimension_semantics=("parallel", "parallel")),
    )(LL, LH, HL, HH, wp, b2)

    cnt = float(N * OHW)
    s = jnp.sum(stats.reshape(N * R, 2, Cout), axis=0)
    mean = s[0] / cnt
    var = jnp.maximum(s[1] / cnt - mean * mean, 0.0)
    scale = (bn_gamma.astype(_f32) * jax.lax.rsqrt(var + eps)).reshape(1, Cout)
    shift = bn_beta.astype(_f32).reshape(1, Cout) - mean.reshape(1, Cout) * scale

    RG = 16 if H2 % 16 == 0 else 1
    R2 = H2 // RG
    L2 = RG * W2
    y_bn = pl.pallas_call(
        _make_bn_kernel(RG, W2),
        out_shape=jax.ShapeDtypeStruct((N, Cout, H2, W2), _f32),
        grid_spec=pltpu.PrefetchScalarGridSpec(
            num_scalar_prefetch=0,
            grid=(N, R2),
            in_specs=[pl.BlockSpec((1, L2, Cout), lambda n, r: (n, r, 0)),
                      pl.BlockSpec((1, Cout), lambda n, r: (0, 0)),
                      pl.BlockSpec((1, Cout), lambda n, r: (0, 0))],
            out_specs=pl.BlockSpec((1, Cout, RG, W2), lambda n, r: (n, 0, r, 0))),
        compiler_params=pltpu.CompilerParams(
            dimension_semantics=("parallel", "parallel")),
    )(y, scale, shift)

    return y_bn


# speedup vs baseline: 1.7488x; 1.0070x over previous
import jax
import jax.numpy as jnp
from jax.experimental import pallas as pl
from jax.experimental.pallas import tpu as pltpu

_f32 = jnp.float32
_bf16 = jnp.bfloat16


def _make_upconv_kernel(H, W, TH):
    def body(ll_ref, lh_ref, hl_ref, hh_ref, wp_ref, b_ref, y_ref, stats_ref):
        Cin = ll_ref.shape[1]
        Cout = wp_ref.shape[4]
        M = TH * W
        SL = (TH + 2) * W

        r = pl.program_id(1)
        nrb = pl.num_programs(1)
        r0 = pl.multiple_of(r * TH, TH)

        def pair_cl(refa, refb, start, n, relu):
            va = refa[0, :, pl.ds(start, n), :].reshape(Cin, n * W)
            if relu:
                va = jnp.maximum(va, 0.0)
            vb = refb[0, :, pl.ds(start, n), :].reshape(Cin, n * W)
            return jnp.transpose(jnp.concatenate([va, vb], axis=0), (1, 0))

        tmask = (r > 0).astype(_f32)
        bmask = (r < nrb - 1).astype(_f32)
        top = jnp.maximum(r0 - 1, 0)
        bot = jnp.minimum(r0 + TH, H - 1)

        def slab(refa, refb, relu):
            t = pair_cl(refa, refb, top, 1, relu) * tmask
            m = pair_cl(refa, refb, r0, TH, relu)
            b = pair_cl(refa, refb, bot, 1, relu) * bmask
            return jnp.concatenate([t, m, b], axis=0)

        t1 = slab(ll_ref, lh_ref, True)
        t2 = slab(hl_ref, hh_ref, False)

        lane = jax.lax.broadcasted_iota(jnp.int32, (SL, 2 * Cin), 1)
        sgn = jnp.where(lane < Cin, -1.0, 1.0).astype(_f32)

        def rot(x):
            return jnp.concatenate([x[:, Cin:], x[:, :Cin]], axis=1)

        u = t1 - t2
        v = t1 + t2
        p0 = (0.5 * (u + sgn * rot(u))).astype(_bf16)
        p1 = (0.5 * (v + sgn * rot(v))).astype(_bf16)

        col = jax.lax.broadcasted_iota(jnp.int32, (SL, 2 * Cin), 0) % W
        ml = ((lane < Cin) & (col != W - 1)).astype(_bf16)
        mr = ((lane >= Cin) & (col != 0)).astype(_bf16)
        zrow = jnp.zeros((1, 2 * Cin), _bf16)

        def shifted(x):
            sp = jnp.concatenate([x[1:], zrow], axis=0)
            sn = jnp.concatenate([zrow, x[:-1]], axis=0)
            return sp * ml + sn * mr

        ps0 = shifted(p0)
        ps1 = shifted(p1)
        pb = (p0, p1)
        psb = (ps0, ps1)

        accs = []
        for p in (0, 1):
            for q in (0, 1):
                acc = jnp.broadcast_to(b_ref[...], (M, Cout))
                for dy in (-1, 0, 1):
                    s = (p + dy) & 1
                    beg = ((p + dy) >> 1) * W + W
                    acc = acc + jnp.dot(pb[s][beg:beg + M],
                                        wp_ref[q, dy + 1, 0],
                                        preferred_element_type=_f32)
                    acc = acc + jnp.dot(psb[s][beg:beg + M],
                                        wp_ref[q, dy + 1, 1],
                                        preferred_element_type=_f32)
                accs.append(acc.reshape(TH, W, Cout))

        even = jnp.stack([accs[0], accs[1]], axis=2)
        odd = jnp.stack([accs[2], accs[3]], axis=2)
        full = jnp.stack([even, odd], axis=1)
        fullb = full.reshape(4 * M, Cout).astype(_bf16)
        y_ref[0] = fullb

        ones = jnp.ones((1, 4 * M), _bf16)
        stats_ref[0, 0, 0:1, :] = jnp.dot(ones, fullb,
                                          preferred_element_type=_f32)
        stats_ref[0, 0, 1:2, :] = jnp.dot(ones, fullb * fullb,
                                          preferred_element_type=_f32)

    return body


def _make_bn_kernel(RG, W2, cnt, eps):
    def body(y_ref, stats_ref, gamma_ref, beta_ref, o_ref):
        s = jnp.sum(stats_ref[...].reshape(-1, 2, stats_ref.shape[-1]), axis=0)
        mean = s[0:1] / cnt
        var = jnp.maximum(s[1:2] / cnt - mean * mean, 0.0)
        scale = gamma_ref[...] * jax.lax.rsqrt(var + eps)
        shift = beta_ref[...] - mean * scale
        y = y_ref[0].astype(_f32) * scale + shift
        t = jnp.transpose(y, (1, 0))
        o_ref[0] = t.reshape(t.shape[0], RG, W2)

    return body


def kernel(LL, LH, HL, HH, conv_w, conv_b, bn_gamma, bn_beta, *, eps=1e-5):
    N, Cin, H, W = LL.shape
    Cout = conv_w.shape[0]
    OHW = 4 * H * W
    H2, W2 = 2 * H, 2 * W
    TH = 16 if H % 16 == 0 else H
    R = H // TH

    wt = jnp.transpose(conv_w, (2, 3, 1, 0))
    z = jnp.zeros((3, Cin, Cout), conv_w.dtype)
    wp = jnp.stack([
        jnp.stack([jnp.concatenate([wt[:, 1], wt[:, 2]], axis=1),
                   jnp.concatenate([z, wt[:, 0]], axis=1)], axis=1),
        jnp.stack([jnp.concatenate([wt[:, 0], wt[:, 1]], axis=1),
                   jnp.concatenate([wt[:, 2], z], axis=1)], axis=1),
    ], axis=0).astype(_bf16)
    b2 = conv_b.reshape(1, Cout).astype(_f32)

    band_spec = pl.BlockSpec((1, Cin, H, W), lambda n, r: (n, 0, 0, 0))

    y, stats = pl.pallas_call(
        _make_upconv_kernel(H, W, TH),
        out_shape=(jax.ShapeDtypeStruct((N, OHW, Cout), _bf16),
                   jax.ShapeDtypeStruct((N, R, 2, Cout), _f32)),
        grid_spec=pltpu.PrefetchScalarGridSpec(
            num_scalar_prefetch=0,
            grid=(N, R),
            in_specs=[band_spec, band_spec, band_spec, band_spec,
                      pl.BlockSpec((2, 3, 2, 2 * Cin, Cout),
                                   lambda n, r: (0, 0, 0, 0, 0)),
                      pl.BlockSpec((1, Cout), lambda n, r: (0, 0))],
            out_specs=(pl.BlockSpec((1, 4 * TH * W, Cout),
                                    lambda n, r: (n, r, 0)),
                       pl.BlockSpec((1, 1, 2, Cout),
                                    lambda n, r: (n, r, 0, 0)))),
        compiler_params=pltpu.CompilerParams(
            dimension_semantics=("parallel", "parallel")),
    )(LL, LH, HL, HH, wp, b2)

    gam = bn_gamma.reshape(1, Cout).astype(_f32)
    bet = bn_beta.reshape(1, Cout).astype(_f32)
    RG = 16 if H2 % 16 == 0 else 1
    R2 = H2 // RG
    L2 = RG * W2
    y_bn = pl.pallas_call(
        _make_bn_kernel(RG, W2, float(N * OHW), eps),
        out_shape=jax.ShapeDtypeStruct((N, Cout, H2, W2), _f32),
        grid_spec=pltpu.PrefetchScalarGridSpec(
            num_scalar_prefetch=0,
            grid=(N, R2),
            in_specs=[pl.BlockSpec((1, L2, Cout), lambda n, r: (n, r, 0)),
                      pl.BlockSpec((N, R, 2, Cout), lambda n, r: (0, 0, 0, 0)),
                      pl.BlockSpec((1, Cout), lambda n, r: (0, 0)),
                      pl.BlockSpec((1, Cout), lambda n, r: (0, 0))],
            out_specs=pl.BlockSpec((1, Cout, RG, W2), lambda n, r: (n, 0, r, 0))),
        compiler_params=pltpu.CompilerParams(
            dimension_semantics=("parallel", "parallel")),
    )(y, stats, gam, bet)

    return y_bn
